# baseline (device time: 2193271 ns/iter reference)
import jax
import jax.numpy as jnp
from jax import lax
from jax.experimental import pallas as pl
from jax.experimental.pallas import tpu as pltpu


NCHUNKS = 8


def kernel(x):
    m_per, n = x.shape
    rows = m_per // NCHUNKS
    x_bf = x.astype(jnp.bfloat16)

    def body(x_ref, out_ref, copy_sem, send_sems, recv_sems):
        my_x = lax.axis_index("x")
        my_y = lax.axis_index("y")
        my_z = lax.axis_index("z")
        nbr = (my_x, my_y, 1 - my_z)

        barrier = pltpu.get_barrier_semaphore()
        pl.semaphore_signal(
            barrier, inc=1, device_id=nbr, device_id_type=pl.DeviceIdType.MESH
        )
        pl.semaphore_wait(barrier, 1)

        base = my_z * m_per

        local = pltpu.make_async_copy(
            x_ref, out_ref.at[pl.ds(base, m_per), :], copy_sem
        )
        local.start()

        rdmas = []
        for c in range(NCHUNKS):
            r = pltpu.make_async_remote_copy(
                src_ref=x_ref.at[pl.ds(c * rows, rows), :],
                dst_ref=out_ref.at[pl.ds(base + c * rows, rows), :],
                send_sem=send_sems.at[c],
                recv_sem=recv_sems.at[c],
                device_id=nbr,
                device_id_type=pl.DeviceIdType.MESH,
            )
            r.start()
            rdmas.append(r)

        local.wait()
        for r in rdmas:
            r.wait()

    return pl.pallas_call(
        body,
        out_shape=jax.ShapeDtypeStruct((2 * m_per, n), jnp.bfloat16),
        in_specs=[pl.BlockSpec(memory_space=pl.ANY)],
        out_specs=pl.BlockSpec(memory_space=pl.ANY),
        scratch_shapes=[
            pltpu.SemaphoreType.DMA,
            pltpu.SemaphoreType.DMA((NCHUNKS,)),
            pltpu.SemaphoreType.DMA((NCHUNKS,)),
        ],
        compiler_params=pltpu.CompilerParams(collective_id=0),
    )(x_bf)


# device time: 875084 ns/iter; 2.5064x vs baseline; 2.5064x over previous
import jax
import jax.numpy as jnp
from jax import lax
from jax.experimental import pallas as pl
from jax.experimental.pallas import tpu as pltpu

VROWS = 4096


def kernel(x):
    m_per, n = x.shape
    nch = m_per // VROWS
    x_bf = x.astype(jnp.bfloat16)

    def body(x_ref, out_ref, vbuf, load_sems, store_sems, send_sem, recv_sem):
        my_x = lax.axis_index("x")
        my_y = lax.axis_index("y")
        my_z = lax.axis_index("z")
        nbr = (my_x, my_y, 1 - my_z)

        barrier = pltpu.get_barrier_semaphore()
        pl.semaphore_signal(
            barrier, inc=1, device_id=nbr, device_id_type=pl.DeviceIdType.MESH
        )
        pl.semaphore_wait(barrier, 1)

        base = my_z * m_per

        rdma = pltpu.make_async_remote_copy(
            src_ref=x_ref,
            dst_ref=out_ref.at[pl.ds(base, m_per), :],
            send_sem=send_sem,
            recv_sem=recv_sem,
            device_id=nbr,
            device_id_type=pl.DeviceIdType.MESH,
        )
        rdma.start()

        stores = []
        for c in range(nch):
            slot = c % 2
            if c >= 2:
                stores[c - 2].wait()
            ld = pltpu.make_async_copy(
                x_ref.at[pl.ds(c * VROWS, VROWS), :],
                vbuf.at[slot],
                load_sems.at[slot],
            )
            ld.start()
            ld.wait()
            st = pltpu.make_async_copy(
                vbuf.at[slot],
                out_ref.at[pl.ds(base + c * VROWS, VROWS), :],
                store_sems.at[slot],
            )
            st.start()
            stores.append(st)
        stores[-2].wait()
        stores[-1].wait()

        rdma.wait()

    return pl.pallas_call(
        body,
        out_shape=jax.ShapeDtypeStruct((2 * m_per, n), jnp.bfloat16),
        in_specs=[pl.BlockSpec(memory_space=pl.ANY)],
        out_specs=pl.BlockSpec(memory_space=pl.ANY),
        scratch_shapes=[
            pltpu.VMEM((2, VROWS, n), jnp.bfloat16),
            pltpu.SemaphoreType.DMA((2,)),
            pltpu.SemaphoreType.DMA((2,)),
            pltpu.SemaphoreType.DMA,
            pltpu.SemaphoreType.DMA,
        ],
        compiler_params=pltpu.CompilerParams(collective_id=0),
    )(x_bf)


# device time: 811274 ns/iter; 2.7035x vs baseline; 1.0787x over previous
import jax
import jax.numpy as jnp
from jax import lax
from jax.experimental import pallas as pl
from jax.experimental.pallas import tpu as pltpu

VROWS = 1024
NFSLOTS = 2
NBSLOTS = 4


def kernel(x):
    m_per, n = x.shape
    nch = m_per // VROWS

    def body(x_ref, out_ref, fbuf, bbuf, load_sems, store_sems, send_sems, recv_sems):
        my_x = lax.axis_index("x")
        my_y = lax.axis_index("y")
        my_z = lax.axis_index("z")
        nbr = (my_x, my_y, 1 - my_z)

        barrier = pltpu.get_barrier_semaphore()
        pl.semaphore_signal(
            barrier, inc=1, device_id=nbr, device_id_type=pl.DeviceIdType.MESH
        )
        pl.semaphore_wait(barrier, 1)

        base = my_z * m_per

        stores = []
        rdmas = []
        for c in range(nch):
            fslot = c % NFSLOTS
            bslot = c % NBSLOTS

            ld = pltpu.make_async_copy(
                x_ref.at[pl.ds(c * VROWS, VROWS), :],
                fbuf.at[fslot],
                load_sems.at[fslot],
            )
            ld.start()
            ld.wait()

            if c >= NBSLOTS:
                rdmas[c - NBSLOTS].wait_send()
                stores[c - NBSLOTS].wait()

            bbuf[bslot] = fbuf[fslot][...].astype(jnp.bfloat16)

            st = pltpu.make_async_copy(
                bbuf.at[bslot],
                out_ref.at[pl.ds(base + c * VROWS, VROWS), :],
                store_sems.at[bslot],
            )
            st.start()
            stores.append(st)

            r = pltpu.make_async_remote_copy(
                src_ref=bbuf.at[bslot],
                dst_ref=out_ref.at[pl.ds(base + c * VROWS, VROWS), :],
                send_sem=send_sems.at[bslot],
                recv_sem=recv_sems.at[c],
                device_id=nbr,
                device_id_type=pl.DeviceIdType.MESH,
            )
            r.start()
            rdmas.append(r)

        for c in range(nch - NBSLOTS, nch):
            rdmas[c].wait_send()
            stores[c].wait()
        for c in range(nch):
            rdmas[c].wait_recv()

    return pl.pallas_call(
        body,
        out_shape=jax.ShapeDtypeStruct((2 * m_per, n), jnp.bfloat16),
        in_specs=[pl.BlockSpec(memory_space=pl.ANY)],
        out_specs=pl.BlockSpec(memory_space=pl.ANY),
        scratch_shapes=[
            pltpu.VMEM((NFSLOTS, VROWS, n), jnp.float32),
            pltpu.VMEM((NBSLOTS, VROWS, n), jnp.bfloat16),
            pltpu.SemaphoreType.DMA((NFSLOTS,)),
            pltpu.SemaphoreType.DMA((NBSLOTS,)),
            pltpu.SemaphoreType.DMA((NBSLOTS,)),
            pltpu.SemaphoreType.DMA((nch,)),
        ],
        compiler_params=pltpu.CompilerParams(collective_id=0),
    )(x)
